# baseline (speedup 1.0000x reference)
import functools

import jax
import jax.numpy as jnp
from jax import lax
from jax.experimental import pallas as pl
from jax.experimental.pallas import tpu as pltpu
from jax.experimental.pallas import tpu_sc as plsc

N = 10000
D = 128
E = 320000
NC, NS = 2, 16
NW = NC * NS
CHUNK = 128
EPW = 10240
EPAD = NW * EPW
NCHUNK = EPW // CHUNK
NP = 10240
RPW = NP // NS
NPA = 10112
RPA = NPA // NS
NCH = EPAD // CHUNK
CPA = 81
NCHA = NW * CPA
EPA = NCHA * CHUNK
NBUF = 3
BM = 2048
NBLK = NP // BM

_MESH = dict(core_axis_name="c", subcore_axis_name="s", num_cores=NC,
             num_subcores=NS)


def _sc_deg(dst_r):
    @functools.partial(
        pl.kernel,
        out_type=jax.ShapeDtypeStruct((NC, NP), jnp.float32),
        mesh=plsc.VectorSubcoreMesh(**_MESH),
        scratch_types=[
            pltpu.VMEM((NCHUNK, CHUNK), jnp.int32),
            pltpu.VMEM((CHUNK,), jnp.float32),
            pltpu.VMEM((RPW,), jnp.float32),
            pltpu.VMEM_SHARED((NP,), jnp.float32),
        ],
    )
    def k(dst_hbm, out_hbm, didx, ones_v, zbuf, acc):
        c = lax.axis_index("c")
        s = lax.axis_index("s")
        w = s * NC + c
        base = w * NCHUNK

        def fill0(i, _):
            zbuf[pl.ds(i * 16, 16)] = jnp.zeros((16,), jnp.float32)
            return 0

        lax.fori_loop(0, RPW // 16, fill0, 0)

        def fill1(i, _):
            ones_v[pl.ds(i * 16, 16)] = jnp.ones((16,), jnp.float32)
            return 0

        lax.fori_loop(0, CHUNK // 16, fill1, 0)

        pltpu.sync_copy(zbuf, acc.at[pl.ds(s * RPW, RPW)])
        pltpu.sync_copy(dst_hbm.at[pl.ds(base, NCHUNK)], didx)
        plsc.subcore_barrier()

        def body(j, _):
            pltpu.sync_copy(ones_v, acc.at[didx.at[j]], add=True)
            return 0

        lax.fori_loop(0, NCHUNK, body, 0)
        plsc.subcore_barrier()
        pltpu.sync_copy(acc.at[pl.ds(s * RPW, RPW)],
                        out_hbm.at[c, pl.ds(s * RPW, RPW)])

    return k(dst_r)


def _sc_agg(g_pad, src_r, dst_r, zblk):
    @functools.partial(
        pl.kernel,
        out_type=jax.ShapeDtypeStruct((NC, NP, D), jnp.float32),
        mesh=plsc.VectorSubcoreMesh(**_MESH),
        scratch_types=[
            [pltpu.VMEM((CHUNK,), jnp.int32) for _ in range(NBUF)],
            [pltpu.VMEM((CHUNK,), jnp.int32) for _ in range(NBUF)],
            [pltpu.VMEM((CHUNK, D), jnp.float32) for _ in range(NBUF)],
            pltpu.VMEM_SHARED((NPA, D), jnp.float32),
            [pltpu.SemaphoreType.DMA for _ in range(NBUF)],
            [pltpu.SemaphoreType.DMA for _ in range(NBUF)],
            [pltpu.SemaphoreType.DMA for _ in range(NBUF)],
            [pltpu.SemaphoreType.DMA for _ in range(NBUF)],
        ],
    )
    def k(g_hbm, src_hbm, dst_hbm, z_hbm, out_hbm,
          sbufs, dbufs, rbufs, acc, semsrc, semdst, semg, semsc):
        c = lax.axis_index("c")
        s = lax.axis_index("s")
        w = s * NC + c
        base = w * CPA

        pltpu.sync_copy(z_hbm, acc.at[pl.ds(s * RPA, RPA)])
        plsc.subcore_barrier()

        def sstart(j, u):
            pltpu.async_copy(src_hbm.at[j], sbufs[u], semsrc[u])

        def swait(j, u):
            pltpu.make_async_copy(src_hbm.at[j], sbufs[u], semsrc[u]).wait()

        def dstart(j, u):
            pltpu.async_copy(dst_hbm.at[j], dbufs[u], semdst[u])

        def dwait(j, u):
            pltpu.make_async_copy(dst_hbm.at[j], dbufs[u], semdst[u]).wait()

        def gstart(u):
            pltpu.async_copy(g_hbm.at[sbufs[u]], rbufs[u], semg[u])

        def gwait(u):
            pltpu.make_async_copy(g_hbm.at[sbufs[u]], rbufs[u],
                                  semg[u]).wait()

        def scstart(u):
            pltpu.async_copy(rbufs[u], acc.at[dbufs[u]], semsc[u], add=True)

        def scwait(u):
            pltpu.make_async_copy(rbufs[u], acc.at[dbufs[u]],
                                  semsc[u]).wait()

        for u in range(NBUF):
            sstart(base + u, u)
            dstart(base + u, u)
        for u in range(NBUF):
            swait(base + u, u)
            gstart(u)

        NB = CPA // NBUF

        def body(t, _):
            j0 = base + t * NBUF
            for u in range(NBUF):
                gwait(u)

                @pl.when(t < NB - 1)
                def _():
                    sstart(j0 + u + NBUF, u)

                dwait(j0 + u, u)
                scstart(u)
            for u in range(NBUF):
                @pl.when(t < NB - 1)
                def _():
                    scwait(u)
                    dstart(j0 + u + NBUF, u)
                    swait(j0 + u + NBUF, u)
                    gstart(u)
            return 0

        lax.fori_loop(0, NB, body, 0)
        for u in range(NBUF):
            scwait(u)
        plsc.subcore_barrier()
        pltpu.sync_copy(acc.at[pl.ds(s * RPA, RPA)],
                        out_hbm.at[c, pl.ds(s * RPA, RPA)])

    return k(g_pad, src_r, dst_r, zblk)


def _tc_g(x_pad, W, deg_part):
    def body(x_ref, w_ref, deg_ref, g_ref):
        deg = deg_ref[0, :] + deg_ref[1, :] + 1.0
        dinv = lax.rsqrt(deg)[:, None]
        g_ref[...] = jnp.dot(x_ref[...], w_ref[...],
                             preferred_element_type=jnp.float32) * dinv

    return pl.pallas_call(
        body,
        grid=(NBLK,),
        in_specs=[
            pl.BlockSpec((BM, D), lambda i: (i, 0)),
            pl.BlockSpec((D, D), lambda i: (0, 0)),
            pl.BlockSpec((NC, BM), lambda i: (0, i)),
        ],
        out_specs=pl.BlockSpec((BM, D), lambda i: (i, 0)),
        out_shape=jax.ShapeDtypeStruct((NP, D), jnp.float32),
    )(x_pad, W, deg_part)


def _tc_pre(agg_part, g, deg_part, bias2d):
    def body(a_ref, g_ref, deg_ref, b_ref, pre_ref, st_ref, accv):
        i = pl.program_id(0)
        deg = deg_ref[0, :] + deg_ref[1, :] + 1.0
        dinv = lax.rsqrt(deg)[:, None]
        pre = (a_ref[0] + a_ref[1] + g_ref[...]) * dinv + b_ref[...]
        pre_ref[...] = pre
        rid = lax.broadcasted_iota(jnp.int32, (BM, 1), 0) + i * BM
        pz = jnp.where(rid < N, pre, 0.0)

        @pl.when(i == 0)
        def _():
            accv[...] = jnp.zeros_like(accv)

        accv[0, :] += jnp.sum(pz, axis=0)
        accv[1, :] += jnp.sum(pz * pz, axis=0)

        @pl.when(i == NBLK - 1)
        def _():
            st_ref[...] = accv[...]

    return pl.pallas_call(
        body,
        grid=(NBLK,),
        in_specs=[
            pl.BlockSpec((NC, BM, D), lambda i: (0, i, 0)),
            pl.BlockSpec((BM, D), lambda i: (i, 0)),
            pl.BlockSpec((NC, BM), lambda i: (0, i)),
            pl.BlockSpec((1, D), lambda i: (0, 0)),
        ],
        out_specs=[
            pl.BlockSpec((BM, D), lambda i: (i, 0)),
            pl.BlockSpec((2, D), lambda i: (0, 0)),
        ],
        out_shape=[
            jax.ShapeDtypeStruct((NP, D), jnp.float32),
            jax.ShapeDtypeStruct((2, D), jnp.float32),
        ],
        scratch_shapes=[pltpu.VMEM((2, D), jnp.float32)],
    )(agg_part, g, deg_part, bias2d)


def _tc_norm(pre, stats, gamma2d, beta2d):
    def body(p_ref, st_ref, gm_ref, bt_ref, o_ref):
        mean = st_ref[0, :] * (1.0 / N)
        var = st_ref[1, :] * (1.0 / N) - mean * mean
        scale = lax.rsqrt(var + 1e-5) * gm_ref[0, :]
        o_ref[...] = jnp.maximum((p_ref[...] - mean) * scale + bt_ref[0, :],
                                 0.0)

    return pl.pallas_call(
        body,
        grid=(NBLK,),
        in_specs=[
            pl.BlockSpec((BM, D), lambda i: (i, 0)),
            pl.BlockSpec((2, D), lambda i: (0, 0)),
            pl.BlockSpec((1, D), lambda i: (0, 0)),
            pl.BlockSpec((1, D), lambda i: (0, 0)),
        ],
        out_specs=pl.BlockSpec((BM, D), lambda i: (i, 0)),
        out_shape=jax.ShapeDtypeStruct((NP, D), jnp.float32),
    )(pre, stats, gamma2d, beta2d)


def kernel(x, edge_index, W, bias, gamma, beta):
    src = edge_index[0].astype(jnp.int32)
    dst = edge_index[1].astype(jnp.int32)
    dst_deg = jnp.concatenate(
        [dst, jnp.full((EPAD - E,), N, jnp.int32)]).reshape(NCH, CHUNK)
    src_agg = jnp.concatenate(
        [src, jnp.zeros((EPA - E,), jnp.int32)]).reshape(NCHA, CHUNK)
    dst_agg = jnp.concatenate(
        [dst, jnp.full((EPA - E,), N, jnp.int32)]).reshape(NCHA, CHUNK)
    x_pad = jnp.pad(x, ((0, NP - N), (0, 0)))
    zblk = jnp.zeros((RPA, D), jnp.float32)

    deg_part = _sc_deg(dst_deg)
    g = _tc_g(x_pad, W, deg_part)
    agg_part = _sc_agg(g, src_agg, dst_agg, zblk)
    pre, stats = _tc_pre(agg_part, g, deg_part, bias.reshape(1, D))
    out = _tc_norm(pre, stats, gamma.reshape(1, D), beta.reshape(1, D))
    return out[:N]


# speedup vs baseline: 1.3647x; 1.3647x over previous
import functools

import jax
import jax.numpy as jnp
from jax import lax
from jax.experimental import pallas as pl
from jax.experimental.pallas import tpu as pltpu
from jax.experimental.pallas import tpu_sc as plsc

N = 10000
D = 128
E = 320000
NC, NS = 2, 16
NW = NC * NS
CHUNK = 128
EPW = 10240
EPAD = NW * EPW
NCHUNK = EPW // CHUNK
NP = 10240
RPW = NP // NS
NPA = 10112
RPA = NPA // NS
NCH = EPAD // CHUNK
BM = 2048
NBLK = NP // BM

_MESH = dict(core_axis_name="c", subcore_axis_name="s", num_cores=NC,
             num_subcores=NS)


def _sc_deg(dst_r):
    @functools.partial(
        pl.kernel,
        out_type=jax.ShapeDtypeStruct((NC, NP), jnp.float32),
        mesh=plsc.VectorSubcoreMesh(**_MESH),
        scratch_types=[
            pltpu.VMEM((NCHUNK, CHUNK), jnp.int32),
            pltpu.VMEM((CHUNK,), jnp.float32),
            pltpu.VMEM((RPW,), jnp.float32),
            pltpu.VMEM_SHARED((NP,), jnp.float32),
        ],
    )
    def k(dst_hbm, out_hbm, didx, ones_v, zbuf, acc):
        c = lax.axis_index("c")
        s = lax.axis_index("s")
        w = s * NC + c
        base = w * NCHUNK

        def fill0(i, _):
            zbuf[pl.ds(i * 16, 16)] = jnp.zeros((16,), jnp.float32)
            return 0

        lax.fori_loop(0, RPW // 16, fill0, 0)

        def fill1(i, _):
            ones_v[pl.ds(i * 16, 16)] = jnp.ones((16,), jnp.float32)
            return 0

        lax.fori_loop(0, CHUNK // 16, fill1, 0)

        pltpu.sync_copy(zbuf, acc.at[pl.ds(s * RPW, RPW)])
        pltpu.sync_copy(dst_hbm.at[pl.ds(base, NCHUNK)], didx)
        plsc.subcore_barrier()

        def body(j, _):
            pltpu.sync_copy(ones_v, acc.at[didx.at[j]], add=True)
            return 0

        lax.fori_loop(0, NCHUNK, body, 0)
        plsc.subcore_barrier()
        pltpu.sync_copy(acc.at[pl.ds(s * RPW, RPW)],
                        out_hbm.at[c, pl.ds(s * RPW, RPW)])

    return k(dst_r)


def _sc_agg(g_pad, src_r, dst_r, zblk):
    @functools.partial(
        pl.kernel,
        out_type=jax.ShapeDtypeStruct((NC, NP, D), jnp.float32),
        mesh=plsc.VectorSubcoreMesh(**_MESH),
        scratch_types=[
            pltpu.VMEM((NCHUNK, CHUNK), jnp.int32),
            pltpu.VMEM((CHUNK,), jnp.int32),
            pltpu.VMEM((CHUNK,), jnp.int32),
            pltpu.VMEM((CHUNK, D), jnp.float32),
            pltpu.VMEM((CHUNK, D), jnp.float32),
            pltpu.VMEM_SHARED((NPA, D), jnp.float32),
            pltpu.SemaphoreType.DMA,
            pltpu.SemaphoreType.DMA,
            pltpu.SemaphoreType.DMA,
            pltpu.SemaphoreType.DMA,
        ],
    )
    def k(g_hbm, src_hbm, dst_hbm, z_hbm, out_hbm,
          sidx, d0, d1, rows0, rows1, acc, semg0, semg1, semd0, semd1):
        c = lax.axis_index("c")
        s = lax.axis_index("s")
        w = s * NC + c
        base = w * NCHUNK

        pltpu.sync_copy(z_hbm, acc.at[pl.ds(s * RPA, RPA)])

        def gstart(j, rows, sem):
            pltpu.async_copy(g_hbm.at[sidx.at[j]], rows, sem)

        def gwait(j, rows, sem):
            pltpu.make_async_copy(g_hbm.at[sidx.at[j]], rows, sem).wait()

        def dstart(row, dbuf, sem):
            pltpu.async_copy(dst_hbm.at[row], dbuf, sem)

        def dwait(row, dbuf, sem):
            pltpu.make_async_copy(dst_hbm.at[row], dbuf, sem).wait()

        def scat(rows, dbuf):
            pltpu.sync_copy(rows, acc.at[dbuf], add=True)

        pltpu.sync_copy(src_hbm.at[pl.ds(base, NCHUNK)], sidx)
        plsc.subcore_barrier()

        dstart(base, d0, semd0)
        dstart(base + 1, d1, semd1)
        gstart(0, rows0, semg0)

        def body(t, _):
            j0 = t * 2
            gstart(j0 + 1, rows1, semg1)
            gwait(j0, rows0, semg0)
            dwait(base + j0, d0, semd0)
            scat(rows0, d0)

            @pl.when(t < NCHUNK // 2 - 1)
            def _():
                dstart(base + j0 + 2, d0, semd0)
                gstart(j0 + 2, rows0, semg0)

            gwait(j0 + 1, rows1, semg1)
            dwait(base + j0 + 1, d1, semd1)
            scat(rows1, d1)

            @pl.when(t < NCHUNK // 2 - 1)
            def _():
                dstart(base + j0 + 3, d1, semd1)

            return 0

        lax.fori_loop(0, NCHUNK // 2, body, 0)
        plsc.subcore_barrier()
        pltpu.sync_copy(acc.at[pl.ds(s * RPA, RPA)],
                        out_hbm.at[c, pl.ds(s * RPA, RPA)])

    return k(g_pad, src_r, dst_r, zblk)


def _tc_g(x_pad, W, deg_part):
    def body(x_ref, w_ref, deg_ref, g_ref):
        deg = deg_ref[0, :] + deg_ref[1, :] + 1.0
        dinv = lax.rsqrt(deg)[:, None]
        g_ref[...] = jnp.dot(x_ref[...], w_ref[...],
                             preferred_element_type=jnp.float32) * dinv

    return pl.pallas_call(
        body,
        grid=(NBLK,),
        in_specs=[
            pl.BlockSpec((BM, D), lambda i: (i, 0)),
            pl.BlockSpec((D, D), lambda i: (0, 0)),
            pl.BlockSpec((NC, BM), lambda i: (0, i)),
        ],
        out_specs=pl.BlockSpec((BM, D), lambda i: (i, 0)),
        out_shape=jax.ShapeDtypeStruct((NP, D), jnp.float32),
    )(x_pad, W, deg_part)


def _tc_pre(agg_part, g, deg_part, bias2d):
    def body(a_ref, g_ref, deg_ref, b_ref, pre_ref, st_ref, accv):
        i = pl.program_id(0)
        deg = deg_ref[0, :] + deg_ref[1, :] + 1.0
        dinv = lax.rsqrt(deg)[:, None]
        pre = (a_ref[0] + a_ref[1] + g_ref[...]) * dinv + b_ref[...]
        pre_ref[...] = pre
        rid = lax.broadcasted_iota(jnp.int32, (BM, 1), 0) + i * BM
        pz = jnp.where(rid < N, pre, 0.0)

        @pl.when(i == 0)
        def _():
            accv[...] = jnp.zeros_like(accv)

        accv[0, :] += jnp.sum(pz, axis=0)
        accv[1, :] += jnp.sum(pz * pz, axis=0)

        @pl.when(i == NBLK - 1)
        def _():
            st_ref[...] = accv[...]

    return pl.pallas_call(
        body,
        grid=(NBLK,),
        in_specs=[
            pl.BlockSpec((NC, BM, D), lambda i: (0, i, 0)),
            pl.BlockSpec((BM, D), lambda i: (i, 0)),
            pl.BlockSpec((NC, BM), lambda i: (0, i)),
            pl.BlockSpec((1, D), lambda i: (0, 0)),
        ],
        out_specs=[
            pl.BlockSpec((BM, D), lambda i: (i, 0)),
            pl.BlockSpec((2, D), lambda i: (0, 0)),
        ],
        out_shape=[
            jax.ShapeDtypeStruct((NP, D), jnp.float32),
            jax.ShapeDtypeStruct((2, D), jnp.float32),
        ],
        scratch_shapes=[pltpu.VMEM((2, D), jnp.float32)],
    )(agg_part, g, deg_part, bias2d)


def _tc_norm(pre, stats, gamma2d, beta2d):
    BN = N // 5

    def body(p_ref, st_ref, gm_ref, bt_ref, o_ref):
        mean = st_ref[0, :] * (1.0 / N)
        var = st_ref[1, :] * (1.0 / N) - mean * mean
        scale = lax.rsqrt(var + 1e-5) * gm_ref[0, :]
        o_ref[...] = jnp.maximum((p_ref[...] - mean) * scale + bt_ref[0, :],
                                 0.0)

    return pl.pallas_call(
        body,
        grid=(5,),
        in_specs=[
            pl.BlockSpec((BN, D), lambda i: (i, 0)),
            pl.BlockSpec((2, D), lambda i: (0, 0)),
            pl.BlockSpec((1, D), lambda i: (0, 0)),
            pl.BlockSpec((1, D), lambda i: (0, 0)),
        ],
        out_specs=pl.BlockSpec((BN, D), lambda i: (i, 0)),
        out_shape=jax.ShapeDtypeStruct((N, D), jnp.float32),
    )(pre, stats, gamma2d, beta2d)


def kernel(x, edge_index, W, bias, gamma, beta):
    src = edge_index[0].astype(jnp.int32)
    dst = edge_index[1].astype(jnp.int32)
    src_r = jnp.concatenate(
        [src, jnp.zeros((EPAD - E,), jnp.int32)]).reshape(NCH, CHUNK)
    dst_r = jnp.concatenate(
        [dst, jnp.full((EPAD - E,), N, jnp.int32)]).reshape(NCH, CHUNK)
    x_pad = jnp.pad(x, ((0, NP - N), (0, 0)))
    zblk = jnp.zeros((RPA, D), jnp.float32)

    deg_part = _sc_deg(dst_r)
    g = _tc_g(x_pad, W, deg_part)
    agg_part = _sc_agg(g, src_r, dst_r, zblk)
    pre, stats = _tc_pre(agg_part, g, deg_part, bias.reshape(1, D))
    return _tc_norm(pre, stats, gamma.reshape(1, D), beta.reshape(1, D))


# speedup vs baseline: 1.3687x; 1.0030x over previous
import functools

import jax
import jax.numpy as jnp
from jax import lax
from jax.experimental import pallas as pl
from jax.experimental.pallas import tpu as pltpu
from jax.experimental.pallas import tpu_sc as plsc

N = 10000
D = 128
E = 320000
NC, NS = 2, 16
NW = NC * NS
CHUNK = 128
EPW = 10240
EPAD = NW * EPW
NCHUNK = EPW // CHUNK
NP = 10240
RPW = NP // NS
NPA = 10112
RPA = NPA // NS
NCH = EPAD // CHUNK
BM = 2048
NBLK = NP // BM

_MESH = dict(core_axis_name="c", subcore_axis_name="s", num_cores=NC,
             num_subcores=NS)


def _sc_deg(dst_r):
    @functools.partial(
        pl.kernel,
        out_type=jax.ShapeDtypeStruct((NC, NP), jnp.float32),
        mesh=plsc.VectorSubcoreMesh(**_MESH),
        scratch_types=[
            pltpu.VMEM((NCHUNK, CHUNK), jnp.int32),
            pltpu.VMEM((CHUNK,), jnp.float32),
            pltpu.VMEM((RPW,), jnp.float32),
            pltpu.VMEM_SHARED((NP,), jnp.float32),
        ],
    )
    def k(dst_hbm, out_hbm, didx, ones_v, zbuf, acc):
        c = lax.axis_index("c")
        s = lax.axis_index("s")
        w = s * NC + c
        base = w * NCHUNK

        def fill0(i, _):
            zbuf[pl.ds(i * 16, 16)] = jnp.zeros((16,), jnp.float32)
            return 0

        lax.fori_loop(0, RPW // 16, fill0, 0)

        def fill1(i, _):
            ones_v[pl.ds(i * 16, 16)] = jnp.ones((16,), jnp.float32)
            return 0

        lax.fori_loop(0, CHUNK // 16, fill1, 0)

        pltpu.sync_copy(zbuf, acc.at[pl.ds(s * RPW, RPW)])
        pltpu.sync_copy(dst_hbm.at[pl.ds(base, NCHUNK)], didx)
        plsc.subcore_barrier()

        def body(j, _):
            pltpu.sync_copy(ones_v, acc.at[didx.at[j]], add=True)
            return 0

        lax.fori_loop(0, NCHUNK, body, 0)
        plsc.subcore_barrier()
        pltpu.sync_copy(acc.at[pl.ds(s * RPW, RPW)],
                        out_hbm.at[c, pl.ds(s * RPW, RPW)])

    return k(dst_r)


def _sc_agg(g_pad, src_r, dst_r):
    @functools.partial(
        pl.kernel,
        out_type=jax.ShapeDtypeStruct((NC, NP, D), jnp.float32),
        mesh=plsc.VectorSubcoreMesh(**_MESH),
        scratch_types=[
            pltpu.VMEM((NCHUNK, CHUNK), jnp.int32),
            pltpu.VMEM((CHUNK,), jnp.int32),
            pltpu.VMEM((CHUNK,), jnp.int32),
            pltpu.VMEM((CHUNK, D), jnp.float32),
            pltpu.VMEM((CHUNK, D), jnp.float32),
            pltpu.VMEM((8, D), jnp.float32),
            pltpu.VMEM_SHARED((NPA, D), jnp.float32),
            pltpu.SemaphoreType.DMA,
            pltpu.SemaphoreType.DMA,
            pltpu.SemaphoreType.DMA,
            pltpu.SemaphoreType.DMA,
        ],
    )
    def k(g_hbm, src_hbm, dst_hbm, out_hbm,
          sidx, d0, d1, rows0, rows1, zrow, acc, semg0, semg1, semd0, semd1):
        c = lax.axis_index("c")
        s = lax.axis_index("s")
        w = s * NC + c
        base = w * NCHUNK

        zv = jnp.zeros((16,), jnp.float32)

        def fillz(i, _):
            zrow[i // 8, pl.ds((i % 8) * 16, 16)] = zv
            return 0

        lax.fori_loop(0, 8 * 8, fillz, 0)

        def zcopy(t, _):
            pltpu.sync_copy(zrow, acc.at[pl.ds(s * RPA + t * 8, 8)])
            return 0

        lax.fori_loop(0, RPA // 8, zcopy, 0)

        def gstart(j, rows, sem):
            pltpu.async_copy(g_hbm.at[sidx.at[j]], rows, sem)

        def gwait(j, rows, sem):
            pltpu.make_async_copy(g_hbm.at[sidx.at[j]], rows, sem).wait()

        def dstart(row, dbuf, sem):
            pltpu.async_copy(dst_hbm.at[row], dbuf, sem)

        def dwait(row, dbuf, sem):
            pltpu.make_async_copy(dst_hbm.at[row], dbuf, sem).wait()

        def scat(rows, dbuf):
            pltpu.sync_copy(rows, acc.at[dbuf], add=True)

        pltpu.sync_copy(src_hbm.at[pl.ds(base, NCHUNK)], sidx)
        plsc.subcore_barrier()

        dstart(base, d0, semd0)
        dstart(base + 1, d1, semd1)
        gstart(0, rows0, semg0)

        def body(t, _):
            j0 = t * 2
            gstart(j0 + 1, rows1, semg1)
            gwait(j0, rows0, semg0)
            dwait(base + j0, d0, semd0)
            scat(rows0, d0)

            @pl.when(t < NCHUNK // 2 - 1)
            def _():
                dstart(base + j0 + 2, d0, semd0)
                gstart(j0 + 2, rows0, semg0)

            gwait(j0 + 1, rows1, semg1)
            dwait(base + j0 + 1, d1, semd1)
            scat(rows1, d1)

            @pl.when(t < NCHUNK // 2 - 1)
            def _():
                dstart(base + j0 + 3, d1, semd1)

            return 0

        lax.fori_loop(0, NCHUNK // 2, body, 0)
        plsc.subcore_barrier()
        pltpu.sync_copy(acc.at[pl.ds(s * RPA, RPA)],
                        out_hbm.at[c, pl.ds(s * RPA, RPA)])

    return k(g_pad, src_r, dst_r)


def _tc_g(x_pad, W, deg_part):
    def body(x_ref, w_ref, deg_ref, g_ref):
        deg = deg_ref[0, :] + deg_ref[1, :] + 1.0
        dinv = lax.rsqrt(deg)[:, None]
        g_ref[...] = jnp.dot(x_ref[...], w_ref[...],
                             preferred_element_type=jnp.float32) * dinv

    return pl.pallas_call(
        body,
        grid=(NBLK,),
        in_specs=[
            pl.BlockSpec((BM, D), lambda i: (i, 0)),
            pl.BlockSpec((D, D), lambda i: (0, 0)),
            pl.BlockSpec((NC, BM), lambda i: (0, i)),
        ],
        out_specs=pl.BlockSpec((BM, D), lambda i: (i, 0)),
        out_shape=jax.ShapeDtypeStruct((NP, D), jnp.float32),
    )(x_pad, W, deg_part)


def _tc_pre(agg_part, g, deg_part, bias2d):
    def body(a_ref, g_ref, deg_ref, b_ref, pre_ref, st_ref, accv):
        i = pl.program_id(0)
        deg = deg_ref[0, :] + deg_ref[1, :] + 1.0
        dinv = lax.rsqrt(deg)[:, None]
        pre = (a_ref[0] + a_ref[1] + g_ref[...]) * dinv + b_ref[...]
        pre_ref[...] = pre
        rid = lax.broadcasted_iota(jnp.int32, (BM, 1), 0) + i * BM
        pz = jnp.where(rid < N, pre, 0.0)

        @pl.when(i == 0)
        def _():
            accv[...] = jnp.zeros_like(accv)

        accv[0, :] += jnp.sum(pz, axis=0)
        accv[1, :] += jnp.sum(pz * pz, axis=0)

        @pl.when(i == NBLK - 1)
        def _():
            st_ref[...] = accv[...]

    return pl.pallas_call(
        body,
        grid=(NBLK,),
        in_specs=[
            pl.BlockSpec((NC, BM, D), lambda i: (0, i, 0)),
            pl.BlockSpec((BM, D), lambda i: (i, 0)),
            pl.BlockSpec((NC, BM), lambda i: (0, i)),
            pl.BlockSpec((1, D), lambda i: (0, 0)),
        ],
        out_specs=[
            pl.BlockSpec((BM, D), lambda i: (i, 0)),
            pl.BlockSpec((2, D), lambda i: (0, 0)),
        ],
        out_shape=[
            jax.ShapeDtypeStruct((NP, D), jnp.float32),
            jax.ShapeDtypeStruct((2, D), jnp.float32),
        ],
        scratch_shapes=[pltpu.VMEM((2, D), jnp.float32)],
    )(agg_part, g, deg_part, bias2d)


def _tc_norm(pre, stats, gamma2d, beta2d):
    BN = N // 5

    def body(p_ref, st_ref, gm_ref, bt_ref, o_ref):
        mean = st_ref[0, :] * (1.0 / N)
        var = st_ref[1, :] * (1.0 / N) - mean * mean
        scale = lax.rsqrt(var + 1e-5) * gm_ref[0, :]
        o_ref[...] = jnp.maximum((p_ref[...] - mean) * scale + bt_ref[0, :],
                                 0.0)

    return pl.pallas_call(
        body,
        grid=(5,),
        in_specs=[
            pl.BlockSpec((BN, D), lambda i: (i, 0)),
            pl.BlockSpec((2, D), lambda i: (0, 0)),
            pl.BlockSpec((1, D), lambda i: (0, 0)),
            pl.BlockSpec((1, D), lambda i: (0, 0)),
        ],
        out_specs=pl.BlockSpec((BN, D), lambda i: (i, 0)),
        out_shape=jax.ShapeDtypeStruct((N, D), jnp.float32),
    )(pre, stats, gamma2d, beta2d)


def kernel(x, edge_index, W, bias, gamma, beta):
    src = edge_index[0].astype(jnp.int32)
    dst = edge_index[1].astype(jnp.int32)
    src_r = jnp.concatenate(
        [src, jnp.zeros((EPAD - E,), jnp.int32)]).reshape(NCH, CHUNK)
    dst_r = jnp.concatenate(
        [dst, jnp.full((EPAD - E,), N, jnp.int32)]).reshape(NCH, CHUNK)
    x_pad = jnp.pad(x, ((0, NP - N), (0, 0)))

    deg_part = _sc_deg(dst_r)
    g = _tc_g(x_pad, W, deg_part)
    agg_part = _sc_agg(g, src_r, dst_r)
    pre, stats = _tc_pre(agg_part, g, deg_part, bias.reshape(1, D))
    return _tc_norm(pre, stats, gamma.reshape(1, D), beta.reshape(1, D))


# speedup vs baseline: 1.3738x; 1.0037x over previous
import functools

import jax
import jax.numpy as jnp
from jax import lax
from jax.experimental import pallas as pl
from jax.experimental.pallas import tpu as pltpu
from jax.experimental.pallas import tpu_sc as plsc

N = 10000
D = 128
E = 320000
NC, NS = 2, 16
NW = NC * NS
CHUNK = 128
EPW = 10240
EPAD = NW * EPW
NCHUNK = EPW // CHUNK
NP = 10240
RPW = NP // NS
NPA = 10240
RPA = NPA // NS
NCH = EPAD // CHUNK
BM = 2048
NBLK = NP // BM

_MESH = dict(core_axis_name="c", subcore_axis_name="s", num_cores=NC,
             num_subcores=NS)


def _sc_deg(dst_r):
    @functools.partial(
        pl.kernel,
        out_type=jax.ShapeDtypeStruct((NC, NP), jnp.float32),
        mesh=plsc.VectorSubcoreMesh(**_MESH),
        scratch_types=[
            pltpu.VMEM((NCHUNK, CHUNK), jnp.int32),
            pltpu.VMEM((CHUNK,), jnp.float32),
            pltpu.VMEM((RPW,), jnp.float32),
            pltpu.VMEM_SHARED((NP,), jnp.float32),
        ],
    )
    def k(dst_hbm, out_hbm, didx, ones_v, zbuf, acc):
        c = lax.axis_index("c")
        s = lax.axis_index("s")
        w = s * NC + c
        base = w * NCHUNK

        def fill0(i, _):
            zbuf[pl.ds(i * 16, 16)] = jnp.zeros((16,), jnp.float32)
            return 0

        lax.fori_loop(0, RPW // 16, fill0, 0)

        def fill1(i, _):
            ones_v[pl.ds(i * 16, 16)] = jnp.ones((16,), jnp.float32)
            return 0

        lax.fori_loop(0, CHUNK // 16, fill1, 0)

        pltpu.sync_copy(zbuf, acc.at[pl.ds(s * RPW, RPW)])
        pltpu.sync_copy(dst_hbm.at[pl.ds(base, NCHUNK)], didx)
        plsc.subcore_barrier()

        def body(j, _):
            pltpu.sync_copy(ones_v, acc.at[didx.at[j]], add=True)
            return 0

        lax.fori_loop(0, NCHUNK, body, 0)
        plsc.subcore_barrier()
        pltpu.sync_copy(acc.at[pl.ds(s * RPW, RPW)],
                        out_hbm.at[c, pl.ds(s * RPW, RPW)])

    return k(dst_r)


def _sc_agg(g_pad, src_r, dst_r):
    @functools.partial(
        pl.kernel,
        out_type=jax.ShapeDtypeStruct((NC, NP, D), jnp.float32),
        mesh=plsc.VectorSubcoreMesh(**_MESH),
        scratch_types=[
            pltpu.VMEM((NCHUNK, CHUNK), jnp.int32),
            pltpu.VMEM((CHUNK,), jnp.int32),
            pltpu.VMEM((CHUNK,), jnp.int32),
            pltpu.VMEM((CHUNK, D), jnp.float32),
            pltpu.VMEM((CHUNK, D), jnp.float32),
            pltpu.VMEM((32, D), jnp.float32),
            pltpu.VMEM_SHARED((NPA, D), jnp.float32),
            pltpu.SemaphoreType.DMA,
            pltpu.SemaphoreType.DMA,
            pltpu.SemaphoreType.DMA,
            pltpu.SemaphoreType.DMA,
        ],
    )
    def k(g_hbm, src_hbm, dst_hbm, out_hbm,
          sidx, d0, d1, rows0, rows1, zrow, acc, semg0, semg1, semd0, semd1):
        c = lax.axis_index("c")
        s = lax.axis_index("s")
        w = s * NC + c
        base = w * NCHUNK

        zv = jnp.zeros((16,), jnp.float32)

        def fillz(i, _):
            zrow[i // 8, pl.ds((i % 8) * 16, 16)] = zv
            return 0

        lax.fori_loop(0, 32 * 8, fillz, 0)

        def zcopy(t, _):
            pltpu.sync_copy(zrow, acc.at[pl.ds(s * RPA + t * 32, 32)])
            return 0

        lax.fori_loop(0, RPA // 32, zcopy, 0)

        def gstart(j, rows, sem):
            pltpu.async_copy(g_hbm.at[sidx.at[j]], rows, sem)

        def gwait(j, rows, sem):
            pltpu.make_async_copy(g_hbm.at[sidx.at[j]], rows, sem).wait()

        def dstart(row, dbuf, sem):
            pltpu.async_copy(dst_hbm.at[row], dbuf, sem)

        def dwait(row, dbuf, sem):
            pltpu.make_async_copy(dst_hbm.at[row], dbuf, sem).wait()

        def scat(rows, dbuf):
            pltpu.sync_copy(rows, acc.at[dbuf], add=True)

        pltpu.sync_copy(src_hbm.at[pl.ds(base, NCHUNK)], sidx)
        plsc.subcore_barrier()

        dstart(base, d0, semd0)
        dstart(base + 1, d1, semd1)
        gstart(0, rows0, semg0)

        def body(t, _):
            j0 = t * 2
            gstart(j0 + 1, rows1, semg1)
            gwait(j0, rows0, semg0)
            dwait(base + j0, d0, semd0)
            scat(rows0, d0)

            @pl.when(t < NCHUNK // 2 - 1)
            def _():
                dstart(base + j0 + 2, d0, semd0)
                gstart(j0 + 2, rows0, semg0)

            gwait(j0 + 1, rows1, semg1)
            dwait(base + j0 + 1, d1, semd1)
            scat(rows1, d1)

            @pl.when(t < NCHUNK // 2 - 1)
            def _():
                dstart(base + j0 + 3, d1, semd1)

            return 0

        lax.fori_loop(0, NCHUNK // 2, body, 0)
        plsc.subcore_barrier()
        pltpu.sync_copy(acc.at[pl.ds(s * RPA, RPA)],
                        out_hbm.at[c, pl.ds(s * RPA, RPA)])

    return k(g_pad, src_r, dst_r)


def _tc_g(x_pad, W, deg_part):
    def body(x_ref, w_ref, deg_ref, g_ref):
        deg = deg_ref[0, :] + deg_ref[1, :] + 1.0
        dinv = lax.rsqrt(deg)[:, None]
        g_ref[...] = jnp.dot(x_ref[...], w_ref[...],
                             preferred_element_type=jnp.float32) * dinv

    return pl.pallas_call(
        body,
        grid=(NBLK,),
        in_specs=[
            pl.BlockSpec((BM, D), lambda i: (i, 0)),
            pl.BlockSpec((D, D), lambda i: (0, 0)),
            pl.BlockSpec((NC, BM), lambda i: (0, i)),
        ],
        out_specs=pl.BlockSpec((BM, D), lambda i: (i, 0)),
        out_shape=jax.ShapeDtypeStruct((NP, D), jnp.float32),
    )(x_pad, W, deg_part)


def _tc_pre(agg_part, g, deg_part, bias2d):
    def body(a_ref, g_ref, deg_ref, b_ref, pre_ref, st_ref, accv):
        i = pl.program_id(0)
        deg = deg_ref[0, :] + deg_ref[1, :] + 1.0
        dinv = lax.rsqrt(deg)[:, None]
        pre = (a_ref[0] + a_ref[1] + g_ref[...]) * dinv + b_ref[...]
        pre_ref[...] = pre
        rid = lax.broadcasted_iota(jnp.int32, (BM, 1), 0) + i * BM
        pz = jnp.where(rid < N, pre, 0.0)

        @pl.when(i == 0)
        def _():
            accv[...] = jnp.zeros_like(accv)

        accv[0, :] += jnp.sum(pz, axis=0)
        accv[1, :] += jnp.sum(pz * pz, axis=0)

        @pl.when(i == NBLK - 1)
        def _():
            st_ref[...] = accv[...]

    return pl.pallas_call(
        body,
        grid=(NBLK,),
        in_specs=[
            pl.BlockSpec((NC, BM, D), lambda i: (0, i, 0)),
            pl.BlockSpec((BM, D), lambda i: (i, 0)),
            pl.BlockSpec((NC, BM), lambda i: (0, i)),
            pl.BlockSpec((1, D), lambda i: (0, 0)),
        ],
        out_specs=[
            pl.BlockSpec((BM, D), lambda i: (i, 0)),
            pl.BlockSpec((2, D), lambda i: (0, 0)),
        ],
        out_shape=[
            jax.ShapeDtypeStruct((NP, D), jnp.float32),
            jax.ShapeDtypeStruct((2, D), jnp.float32),
        ],
        scratch_shapes=[pltpu.VMEM((2, D), jnp.float32)],
    )(agg_part, g, deg_part, bias2d)


def _tc_norm(pre, stats, gamma2d, beta2d):
    BN = N // 5

    def body(p_ref, st_ref, gm_ref, bt_ref, o_ref):
        mean = st_ref[0, :] * (1.0 / N)
        var = st_ref[1, :] * (1.0 / N) - mean * mean
        scale = lax.rsqrt(var + 1e-5) * gm_ref[0, :]
        o_ref[...] = jnp.maximum((p_ref[...] - mean) * scale + bt_ref[0, :],
                                 0.0)

    return pl.pallas_call(
        body,
        grid=(5,),
        in_specs=[
            pl.BlockSpec((BN, D), lambda i: (i, 0)),
            pl.BlockSpec((2, D), lambda i: (0, 0)),
            pl.BlockSpec((1, D), lambda i: (0, 0)),
            pl.BlockSpec((1, D), lambda i: (0, 0)),
        ],
        out_specs=pl.BlockSpec((BN, D), lambda i: (i, 0)),
        out_shape=jax.ShapeDtypeStruct((N, D), jnp.float32),
    )(pre, stats, gamma2d, beta2d)


def kernel(x, edge_index, W, bias, gamma, beta):
    src = edge_index[0].astype(jnp.int32)
    dst = edge_index[1].astype(jnp.int32)
    src_r = jnp.concatenate(
        [src, jnp.zeros((EPAD - E,), jnp.int32)]).reshape(NCH, CHUNK)
    dst_r = jnp.concatenate(
        [dst, jnp.full((EPAD - E,), N, jnp.int32)]).reshape(NCH, CHUNK)
    x_pad = jnp.pad(x, ((0, NP - N), (0, 0)))

    deg_part = _sc_deg(dst_r)
    g = _tc_g(x_pad, W, deg_part)
    agg_part = _sc_agg(g, src_r, dst_r)
    pre, stats = _tc_pre(agg_part, g, deg_part, bias.reshape(1, D))
    return _tc_norm(pre, stats, gamma.reshape(1, D), beta.reshape(1, D))


# speedup vs baseline: 1.5685x; 1.1417x over previous
import functools

import jax
import jax.numpy as jnp
from jax import lax
from jax.experimental import pallas as pl
from jax.experimental.pallas import tpu as pltpu
from jax.experimental.pallas import tpu_sc as plsc

N = 10000
D = 128
E = 320000
NC, NS = 2, 16
NW = NC * NS
CHUNK = 128
EPW = 10240
EPAD = NW * EPW
NCHUNK = EPW // CHUNK
NP = 10240
RPW = NP // NS
NPA = 10240
RPA = NPA // NS
NCH = EPAD // CHUNK
BM = 2048
NBLK = NP // BM

_MESH = dict(core_axis_name="c", subcore_axis_name="s", num_cores=NC,
             num_subcores=NS)


def _sc_deg(dst_r):
    @functools.partial(
        pl.kernel,
        out_type=jax.ShapeDtypeStruct((NC, NP), jnp.float32),
        mesh=plsc.VectorSubcoreMesh(**_MESH),
        scratch_types=[
            pltpu.VMEM((NCHUNK, CHUNK), jnp.int32),
            pltpu.VMEM((CHUNK,), jnp.float32),
            pltpu.VMEM((RPW,), jnp.float32),
            pltpu.VMEM_SHARED((NP,), jnp.float32),
        ],
    )
    def k(dst_hbm, out_hbm, didx, ones_v, zbuf, acc):
        c = lax.axis_index("c")
        s = lax.axis_index("s")
        w = s * NC + c

        def fill0(i, _):
            zbuf[pl.ds(i * 16, 16)] = jnp.zeros((16,), jnp.float32)
            return 0

        lax.fori_loop(0, RPW // 16, fill0, 0)

        def fill1(i, _):
            ones_v[pl.ds(i * 16, 16)] = jnp.ones((16,), jnp.float32)
            return 0

        lax.fori_loop(0, CHUNK // 16, fill1, 0)

        pltpu.sync_copy(zbuf, acc.at[pl.ds(s * RPW, RPW)])
        pltpu.sync_copy(dst_hbm.at[w], didx)
        plsc.subcore_barrier()

        def body(j, _):
            pltpu.sync_copy(ones_v, acc.at[didx.at[j]], add=True)
            return 0

        lax.fori_loop(0, NCHUNK, body, 0)
        plsc.subcore_barrier()
        pltpu.sync_copy(acc.at[pl.ds(s * RPW, RPW)],
                        out_hbm.at[c, pl.ds(s * RPW, RPW)])

    return k(dst_r)


def _sc_agg(g_pad, src_r, dst_r):
    @functools.partial(
        pl.kernel,
        out_type=jax.ShapeDtypeStruct((NC, NP, D), jnp.float32),
        mesh=plsc.VectorSubcoreMesh(**_MESH),
        scratch_types=[
            pltpu.VMEM((NCHUNK, CHUNK), jnp.int32),
            pltpu.VMEM((CHUNK,), jnp.int32),
            pltpu.VMEM((CHUNK,), jnp.int32),
            pltpu.VMEM((CHUNK, D), jnp.float32),
            pltpu.VMEM((CHUNK, D), jnp.float32),
            pltpu.VMEM((8, D), jnp.float32),
            pltpu.VMEM_SHARED((NPA, D), jnp.float32),
            pltpu.SemaphoreType.DMA,
            pltpu.SemaphoreType.DMA,
            pltpu.SemaphoreType.DMA,
            pltpu.SemaphoreType.DMA,
        ],
    )
    def k(g_hbm, src_hbm, dst_hbm, out_hbm,
          sidx, d0, d1, rows0, rows1, zrow, acc, semg0, semg1, semd0, semd1):
        c = lax.axis_index("c")
        s = lax.axis_index("s")
        w = s * NC + c

        zv = jnp.zeros((16,), jnp.float32)

        def fillz(i, _):
            zrow[i // 8, pl.ds((i % 8) * 16, 16)] = zv
            return 0

        lax.fori_loop(0, 8 * 8, fillz, 0)

        def zcopy(t, _):
            pltpu.sync_copy(zrow, acc.at[pl.ds(s * RPA + t * 8, 8)])
            return 0

        lax.fori_loop(0, RPA // 8, zcopy, 0)

        def gstart(j, rows, sem):
            pltpu.async_copy(g_hbm.at[sidx.at[j]], rows, sem)

        def gwait(j, rows, sem):
            pltpu.make_async_copy(g_hbm.at[sidx.at[j]], rows, sem).wait()

        def dstart(j, dbuf, sem):
            pltpu.async_copy(dst_hbm.at[w, j], dbuf, sem)

        def dwait(j, dbuf, sem):
            pltpu.make_async_copy(dst_hbm.at[w, j], dbuf, sem).wait()

        def scat(rows, dbuf):
            pltpu.sync_copy(rows, acc.at[dbuf], add=True)

        pltpu.sync_copy(src_hbm.at[w], sidx)
        plsc.subcore_barrier()

        dstart(0, d0, semd0)
        dstart(1, d1, semd1)
        gstart(0, rows0, semg0)

        def body(t, _):
            j0 = t * 2
            gstart(j0 + 1, rows1, semg1)
            gwait(j0, rows0, semg0)
            dwait(j0, d0, semd0)
            scat(rows0, d0)

            @pl.when(t < NCHUNK // 2 - 1)
            def _():
                dstart(j0 + 2, d0, semd0)
                gstart(j0 + 2, rows0, semg0)

            gwait(j0 + 1, rows1, semg1)
            dwait(j0 + 1, d1, semd1)
            scat(rows1, d1)

            @pl.when(t < NCHUNK // 2 - 1)
            def _():
                dstart(j0 + 3, d1, semd1)

            return 0

        lax.fori_loop(0, NCHUNK // 2, body, 0)
        plsc.subcore_barrier()
        pltpu.sync_copy(acc.at[pl.ds(s * RPA, RPA)],
                        out_hbm.at[c, pl.ds(s * RPA, RPA)])

    return k(g_pad, src_r, dst_r)


def _tc_g(x_pad, W, deg_part):
    def body(x_ref, w_ref, deg_ref, g_ref):
        deg = deg_ref[0, :] + deg_ref[1, :] + 1.0
        dinv = lax.rsqrt(deg)[:, None]
        g_ref[...] = jnp.dot(x_ref[...], w_ref[...],
                             preferred_element_type=jnp.float32) * dinv

    return pl.pallas_call(
        body,
        grid=(NBLK,),
        in_specs=[
            pl.BlockSpec((BM, D), lambda i: (i, 0)),
            pl.BlockSpec((D, D), lambda i: (0, 0)),
            pl.BlockSpec((NC, BM), lambda i: (0, i)),
        ],
        out_specs=pl.BlockSpec((BM, D), lambda i: (i, 0)),
        out_shape=jax.ShapeDtypeStruct((NP, D), jnp.float32),
    )(x_pad, W, deg_part)


def _tc_pre(agg_part, g, deg_part, bias2d):
    def body(a_ref, g_ref, deg_ref, b_ref, pre_ref, st_ref, accv):
        i = pl.program_id(0)
        deg = deg_ref[0, :] + deg_ref[1, :] + 1.0
        dinv = lax.rsqrt(deg)[:, None]
        pre = (a_ref[0] + a_ref[1] + g_ref[...]) * dinv + b_ref[...]
        pre_ref[...] = pre
        rid = lax.broadcasted_iota(jnp.int32, (BM, 1), 0) + i * BM
        pz = jnp.where(rid < N, pre, 0.0)

        @pl.when(i == 0)
        def _():
            accv[...] = jnp.zeros_like(accv)

        accv[0, :] += jnp.sum(pz, axis=0)
        accv[1, :] += jnp.sum(pz * pz, axis=0)

        @pl.when(i == NBLK - 1)
        def _():
            st_ref[...] = accv[...]

    return pl.pallas_call(
        body,
        grid=(NBLK,),
        in_specs=[
            pl.BlockSpec((NC, BM, D), lambda i: (0, i, 0)),
            pl.BlockSpec((BM, D), lambda i: (i, 0)),
            pl.BlockSpec((NC, BM), lambda i: (0, i)),
            pl.BlockSpec((1, D), lambda i: (0, 0)),
        ],
        out_specs=[
            pl.BlockSpec((BM, D), lambda i: (i, 0)),
            pl.BlockSpec((2, D), lambda i: (0, 0)),
        ],
        out_shape=[
            jax.ShapeDtypeStruct((NP, D), jnp.float32),
            jax.ShapeDtypeStruct((2, D), jnp.float32),
        ],
        scratch_shapes=[pltpu.VMEM((2, D), jnp.float32)],
    )(agg_part, g, deg_part, bias2d)


def _tc_norm(pre, stats, gamma2d, beta2d):
    def body(p_ref, st_ref, gm_ref, bt_ref, o_ref):
        mean = st_ref[0, :] * (1.0 / N)
        var = st_ref[1, :] * (1.0 / N) - mean * mean
        scale = lax.rsqrt(var + 1e-5) * gm_ref[0, :]
        o_ref[...] = jnp.maximum((p_ref[...] - mean) * scale + bt_ref[0, :],
                                 0.0)

    return pl.pallas_call(
        body,
        grid=(NBLK,),
        in_specs=[
            pl.BlockSpec((BM, D), lambda i: (i, 0)),
            pl.BlockSpec((2, D), lambda i: (0, 0)),
            pl.BlockSpec((1, D), lambda i: (0, 0)),
            pl.BlockSpec((1, D), lambda i: (0, 0)),
        ],
        out_specs=pl.BlockSpec((BM, D), lambda i: (i, 0)),
        out_shape=jax.ShapeDtypeStruct((NP, D), jnp.float32),
    )(pre, stats, gamma2d, beta2d)


def kernel(x, edge_index, W, bias, gamma, beta):
    src = edge_index[0].astype(jnp.int32)
    dst = edge_index[1].astype(jnp.int32)
    src_r = jnp.concatenate(
        [src, jnp.zeros((EPAD - E,), jnp.int32)]).reshape(NW, NCHUNK, CHUNK)
    dst_r = jnp.concatenate(
        [dst, jnp.full((EPAD - E,), N, jnp.int32)]).reshape(NW, NCHUNK, CHUNK)
    x_pad = jnp.pad(x, ((0, NP - N), (0, 0)))

    deg_part = _sc_deg(dst_r)
    g = _tc_g(x_pad, W, deg_part)
    agg_part = _sc_agg(g, src_r, dst_r)
    pre, stats = _tc_pre(agg_part, g, deg_part, bias.reshape(1, D))
    out = _tc_norm(pre, stats, gamma.reshape(1, D), beta.reshape(1, D))
    return out[:N]


# speedup vs baseline: 1.5816x; 1.0084x over previous
import functools

import jax
import jax.numpy as jnp
from jax import lax
from jax.experimental import pallas as pl
from jax.experimental.pallas import tpu as pltpu
from jax.experimental.pallas import tpu_sc as plsc

N = 10000
D = 128
E = 320000
NC, NS = 2, 16
NW = NC * NS
CHUNK = 128
EPW = 10240
EPAD = NW * EPW
NCHUNK = EPW // CHUNK
NP = 10240
RPW = NP // NS
NPA = 10240
RPA = NPA // NS
NCH = EPAD // CHUNK
BM = 2048
NBLK = NP // BM

_MESH = dict(core_axis_name="c", subcore_axis_name="s", num_cores=NC,
             num_subcores=NS)


def _sc_deg(dst_r):
    @functools.partial(
        pl.kernel,
        out_type=jax.ShapeDtypeStruct((NC, NP), jnp.float32),
        mesh=plsc.VectorSubcoreMesh(**_MESH),
        scratch_types=[
            pltpu.VMEM((NCHUNK, CHUNK), jnp.int32),
            pltpu.VMEM((CHUNK,), jnp.float32),
            pltpu.VMEM((RPW,), jnp.float32),
            pltpu.VMEM_SHARED((NP,), jnp.float32),
        ],
    )
    def k(dst_hbm, out_hbm, didx, ones_v, zbuf, acc):
        c = lax.axis_index("c")
        s = lax.axis_index("s")
        w = s * NC + c

        def fill0(i, _):
            zbuf[pl.ds(i * 16, 16)] = jnp.zeros((16,), jnp.float32)
            return 0

        lax.fori_loop(0, RPW // 16, fill0, 0)

        def fill1(i, _):
            ones_v[pl.ds(i * 16, 16)] = jnp.ones((16,), jnp.float32)
            return 0

        lax.fori_loop(0, CHUNK // 16, fill1, 0)

        pltpu.sync_copy(zbuf, acc.at[pl.ds(s * RPW, RPW)])
        pltpu.sync_copy(dst_hbm.at[w], didx)
        plsc.subcore_barrier()

        def body(j, _):
            pltpu.sync_copy(ones_v, acc.at[didx.at[j]], add=True)
            return 0

        lax.fori_loop(0, NCHUNK, body, 0)
        plsc.subcore_barrier()
        pltpu.sync_copy(acc.at[pl.ds(s * RPW, RPW)],
                        out_hbm.at[c, pl.ds(s * RPW, RPW)])

    return k(dst_r)


def _sc_agg(g_pad, src_r, dst_r):
    @functools.partial(
        pl.kernel,
        out_type=jax.ShapeDtypeStruct((NC, NP, D), jnp.float32),
        mesh=plsc.VectorSubcoreMesh(**_MESH),
        scratch_types=[
            pltpu.VMEM((NCHUNK, CHUNK), jnp.int32),
            pltpu.VMEM((CHUNK,), jnp.int32),
            pltpu.VMEM((CHUNK,), jnp.int32),
            pltpu.VMEM((CHUNK, D), jnp.float32),
            pltpu.VMEM((CHUNK, D), jnp.float32),
            pltpu.VMEM((8, D), jnp.float32),
            pltpu.VMEM_SHARED((NPA, D), jnp.float32),
            pltpu.SemaphoreType.DMA,
            pltpu.SemaphoreType.DMA,
            pltpu.SemaphoreType.DMA,
            pltpu.SemaphoreType.DMA,
        ],
    )
    def k(g_hbm, src_hbm, dst_hbm, out_hbm,
          sidx, d0, d1, rows0, rows1, zrow, acc, semg0, semg1, semd0, semd1):
        c = lax.axis_index("c")
        s = lax.axis_index("s")
        w = s * NC + c

        zv = jnp.zeros((16,), jnp.float32)

        def fillz(i, _):
            zrow[i // 8, pl.ds((i % 8) * 16, 16)] = zv
            return 0

        lax.fori_loop(0, 8 * 8, fillz, 0)

        def zcopy(t, _):
            pltpu.sync_copy(zrow, acc.at[pl.ds(s * RPA + t * 8, 8)])
            return 0

        lax.fori_loop(0, RPA // 8, zcopy, 0)

        def gstart(j, rows, sem):
            pltpu.async_copy(g_hbm.at[sidx.at[j]], rows, sem)

        def gwait(j, rows, sem):
            pltpu.make_async_copy(g_hbm.at[sidx.at[j]], rows, sem).wait()

        def dstart(j, dbuf, sem):
            pltpu.async_copy(dst_hbm.at[w, j], dbuf, sem)

        def dwait(j, dbuf, sem):
            pltpu.make_async_copy(dst_hbm.at[w, j], dbuf, sem).wait()

        def scat(rows, dbuf):
            pltpu.sync_copy(rows, acc.at[dbuf], add=True)

        pltpu.sync_copy(src_hbm.at[w], sidx)
        plsc.subcore_barrier()

        dstart(0, d0, semd0)
        dstart(1, d1, semd1)
        gstart(0, rows0, semg0)

        def body(t, _):
            j0 = t * 2
            gstart(j0 + 1, rows1, semg1)
            gwait(j0, rows0, semg0)
            dwait(j0, d0, semd0)
            scat(rows0, d0)

            @pl.when(t < NCHUNK // 2 - 1)
            def _():
                dstart(j0 + 2, d0, semd0)
                gstart(j0 + 2, rows0, semg0)

            gwait(j0 + 1, rows1, semg1)
            dwait(j0 + 1, d1, semd1)
            scat(rows1, d1)

            @pl.when(t < NCHUNK // 2 - 1)
            def _():
                dstart(j0 + 3, d1, semd1)

            return 0

        lax.fori_loop(0, NCHUNK // 2, body, 0)
        plsc.subcore_barrier()
        pltpu.sync_copy(acc.at[pl.ds(s * RPA, RPA)],
                        out_hbm.at[c, pl.ds(s * RPA, RPA)])

    return k(g_pad, src_r, dst_r)


def _tc_g(x_pad, W, deg_part):
    def body(x_ref, w_ref, deg_ref, g_ref):
        deg = deg_ref[0, :] + deg_ref[1, :] + 1.0
        dinv = lax.rsqrt(deg)[:, None]
        g_ref[...] = jnp.dot(x_ref[...], w_ref[...],
                             preferred_element_type=jnp.float32) * dinv

    return pl.pallas_call(
        body,
        grid=(NBLK,),
        in_specs=[
            pl.BlockSpec((BM, D), lambda i: (i, 0)),
            pl.BlockSpec((D, D), lambda i: (0, 0)),
            pl.BlockSpec((NC, BM), lambda i: (0, i)),
        ],
        out_specs=pl.BlockSpec((BM, D), lambda i: (i, 0)),
        out_shape=jax.ShapeDtypeStruct((NP, D), jnp.float32),
    )(x_pad, W, deg_part)


def _tc_pre(agg_part, g, deg_part, bias2d):
    def body(a_ref, g_ref, deg_ref, b_ref, pre_ref, st_ref, accv):
        i = pl.program_id(0)
        deg = deg_ref[0, :] + deg_ref[1, :] + 1.0
        dinv = lax.rsqrt(deg)[:, None]
        pre = (a_ref[0] + a_ref[1] + g_ref[...]) * dinv + b_ref[...]
        pre_ref[...] = pre
        rid = lax.broadcasted_iota(jnp.int32, (BM, 1), 0) + i * BM
        pz = jnp.where(rid < N, pre, 0.0)

        @pl.when(i == 0)
        def _():
            accv[...] = jnp.zeros_like(accv)

        accv[0, :] += jnp.sum(pz, axis=0)
        accv[1, :] += jnp.sum(pz * pz, axis=0)

        @pl.when(i == NBLK - 1)
        def _():
            st_ref[...] = accv[...]

    return pl.pallas_call(
        body,
        grid=(NBLK,),
        in_specs=[
            pl.BlockSpec((NC, BM, D), lambda i: (0, i, 0)),
            pl.BlockSpec((BM, D), lambda i: (i, 0)),
            pl.BlockSpec((NC, BM), lambda i: (0, i)),
            pl.BlockSpec((1, D), lambda i: (0, 0)),
        ],
        out_specs=[
            pl.BlockSpec((BM, D), lambda i: (i, 0)),
            pl.BlockSpec((2, D), lambda i: (0, 0)),
        ],
        out_shape=[
            jax.ShapeDtypeStruct((NP, D), jnp.float32),
            jax.ShapeDtypeStruct((2, D), jnp.float32),
        ],
        scratch_shapes=[pltpu.VMEM((2, D), jnp.float32)],
    )(agg_part, g, deg_part, bias2d)


def _tc_norm(pre, stats, gamma2d, beta2d):
    BN = N // 5

    def body(p_ref, st_ref, gm_ref, bt_ref, o_ref):
        mean = st_ref[0, :] * (1.0 / N)
        var = st_ref[1, :] * (1.0 / N) - mean * mean
        scale = lax.rsqrt(var + 1e-5) * gm_ref[0, :]
        o_ref[...] = jnp.maximum((p_ref[...] - mean) * scale + bt_ref[0, :],
                                 0.0)

    return pl.pallas_call(
        body,
        grid=(5,),
        in_specs=[
            pl.BlockSpec((BN, D), lambda i: (i, 0)),
            pl.BlockSpec((2, D), lambda i: (0, 0)),
            pl.BlockSpec((1, D), lambda i: (0, 0)),
            pl.BlockSpec((1, D), lambda i: (0, 0)),
        ],
        out_specs=pl.BlockSpec((BN, D), lambda i: (i, 0)),
        out_shape=jax.ShapeDtypeStruct((N, D), jnp.float32),
    )(pre, stats, gamma2d, beta2d)


def kernel(x, edge_index, W, bias, gamma, beta):
    src = edge_index[0].astype(jnp.int32)
    dst = edge_index[1].astype(jnp.int32)
    src_r = jnp.concatenate(
        [src, jnp.zeros((EPAD - E,), jnp.int32)]).reshape(NW, NCHUNK, CHUNK)
    dst_r = jnp.concatenate(
        [dst, jnp.full((EPAD - E,), N, jnp.int32)]).reshape(NW, NCHUNK, CHUNK)
    x_pad = jnp.pad(x, ((0, NP - N), (0, 0)))

    deg_part = _sc_deg(dst_r)
    g = _tc_g(x_pad, W, deg_part)
    agg_part = _sc_agg(g, src_r, dst_r)
    pre, stats = _tc_pre(agg_part, g, deg_part, bias.reshape(1, D))
    return _tc_norm(pre, stats, gamma.reshape(1, D), beta.reshape(1, D))
